# baseline (speedup 1.0000x reference)
import functools

import jax
import jax.numpy as jnp
from jax.experimental import pallas as pl


def _fused_kernel(x_ref, w_ref, b_ref, o_ref):
    acc = jnp.dot(x_ref[...], w_ref[...], preferred_element_type=jnp.float32)
    o_ref[...] = jnp.maximum(acc + b_ref[...], 0.0)


@functools.partial(jax.jit, static_argnames=("block_m",))
def _run(inputs, weights, bias2d, block_m=512):
    m, k = inputs.shape
    n = weights.shape[1]
    return pl.pallas_call(
        _fused_kernel,
        grid=(m // block_m,),
        in_specs=[
            pl.BlockSpec((block_m, k), lambda i: (i, 0)),
            pl.BlockSpec((k, n), lambda i: (0, 0)),
            pl.BlockSpec((1, n), lambda i: (0, 0)),
        ],
        out_specs=pl.BlockSpec((block_m, n), lambda i: (i, 0)),
        out_shape=jax.ShapeDtypeStruct((m, n), jnp.float32),
    )(inputs, weights, bias2d)


def kernel(inputs, kernel, bias):
    return _run(inputs, kernel, bias.reshape(1, -1))


# speedup vs baseline: 1.1463x; 1.1463x over previous
import functools

import jax
import jax.numpy as jnp
from jax.experimental import pallas as pl


def _fused_kernel(x_ref, w_ref, b_ref, o_ref):
    acc = jnp.dot(x_ref[...], w_ref[...], preferred_element_type=jnp.float32)
    o_ref[...] = jnp.maximum(acc + b_ref[...], 0.0)


@functools.partial(jax.jit, static_argnames=("block_m",))
def _run(inputs, weights, bias2d, block_m=512):
    m, k = inputs.shape
    n = weights.shape[1]
    return pl.pallas_call(
        _fused_kernel,
        grid=(m // block_m,),
        in_specs=[
            pl.BlockSpec((block_m, k), lambda i: (i, 0)),
            pl.BlockSpec((k, n), lambda i: (0, 0)),
            pl.BlockSpec((1, n), lambda i: (0, 0)),
        ],
        out_specs=pl.BlockSpec((block_m, n), lambda i: (i, 0)),
        out_shape=jax.ShapeDtypeStruct((m, n), jnp.float32),
    )(inputs, weights, bias2d)


def kernel(inputs, kernel, bias):
    return _run(inputs, kernel, bias.reshape(1, -1), block_m=2048)


# speedup vs baseline: 3.6108x; 3.1501x over previous
import jax
import jax.numpy as jnp
from jax.experimental import pallas as pl


def _fused_kernel_t(xt_ref, w_ref, b_ref, o_ref):
    acc = jax.lax.dot_general(
        xt_ref[...], w_ref[...], (((0,), (0,)), ((), ())),
        preferred_element_type=jnp.float32,
    )
    o_ref[...] = jnp.maximum(acc + b_ref[...], 0.0)


@jax.jit
def _run(inputs, weights, bias2d):
    m, k = inputs.shape
    n = weights.shape[1]
    xt = inputs.T
    bn = 2048
    return pl.pallas_call(
        _fused_kernel_t,
        grid=(m // bn,),
        in_specs=[
            pl.BlockSpec((k, bn), lambda i: (0, i)),
            pl.BlockSpec((k, n), lambda i: (0, 0)),
            pl.BlockSpec((1, n), lambda i: (0, 0)),
        ],
        out_specs=pl.BlockSpec((bn, n), lambda i: (i, 0)),
        out_shape=jax.ShapeDtypeStruct((m, n), jnp.float32),
    )(xt, weights, bias2d)


def kernel(inputs, kernel, bias):
    return _run(inputs, kernel, bias.reshape(1, -1))
